# baseline (speedup 1.0000x reference)
import functools

import numpy as _np

import jax
import jax.numpy as jnp
from jax import lax
from jax.experimental import pallas as pl
from jax.experimental.pallas import tpu as pltpu
from jax.experimental.pallas import tpu_sc as plsc

NUM_CLASSES = 1024
FEAT = 256
ALPHA = 0.5
B = 16384

NC, NS = 2, 16
L = 16
HALF = B // NC
CPT = NUM_CLASSES // NS
CHUNK = 128
GRP = HALF // L
NCHK = HALF // CHUNK
LPAD = HALF + CHUNK + L
CPAD = 16
FG = FEAT // L
NSQ = 4

_mesh = plsc.VectorSubcoreMesh(core_axis_name="c", subcore_axis_name="s")


@functools.partial(
    pl.kernel,
    out_type=(
        jax.ShapeDtypeStruct((NC, NUM_CLASSES, FEAT), jnp.float32),
        jax.ShapeDtypeStruct((NC, NUM_CLASSES, CPAD), jnp.float32),
        jax.ShapeDtypeStruct((NC, NS, L), jnp.float32),
    ),
    mesh=_mesh,
    compiler_params=pltpu.CompilerParams(needs_layout_passes=False),
    scratch_types=[
        pltpu.VMEM((HALF,), jnp.int32),
        pltpu.VMEM((NCHK + 1, CHUNK), jnp.int32),
        pltpu.VMEM((LPAD,), jnp.int32),
        pltpu.VMEM((CHUNK, FEAT), jnp.float32),
        pltpu.VMEM((CHUNK, FEAT), jnp.float32),
        pltpu.VMEM((CPT, FEAT), jnp.float32),
        pltpu.VMEM((CPT, CPAD), jnp.float32),
        pltpu.VMEM((NSQ, L), jnp.float32),
        pltpu.SemaphoreType.DMA,
        pltpu.SemaphoreType.DMA,
    ],
)
def _segsum_sc(labels_hbm, ypred_hbm, zrows_hbm, zcnt_hbm,
               sacc_hbm, cacc_hbm, sqacc_hbm,
               lab_v, rid_v, cls_v, rows_a, rows_b, acc_v, cnt_v, sq_v,
               sem_a, sem_b):
    c = lax.axis_index("c")
    s = lax.axis_index("s")
    i32 = jnp.int32
    lo = s * i32(CPT)
    base = c * i32(HALF)

    pltpu.sync_copy(zrows_hbm, acc_v)
    pltpu.sync_copy(zcnt_hbm, cnt_v)
    pltpu.sync_copy(labels_hbm.at[c], lab_v)

    lane = lax.iota(jnp.int32, L)
    one_col = jnp.where(lane == 0, jnp.float32(1.0), jnp.float32(0.0))
    zvec = jnp.zeros((L,), jnp.float32)
    for q in range(NSQ):
        sq_v[i32(q), pl.ds(0, L)] = zvec

    def scan_body(g, n):
        lbl = lab_v[pl.ds(g * i32(L), L)]
        in_stripe = jnp.logical_and(lbl >= lo, lbl < lo + CPT)
        rows = base + g * i32(L) + lane
        inc = plsc.cumsum(in_stripe.astype(jnp.int32))
        pos = jnp.maximum(n + inc - 1, i32(0))
        prow = lax.shift_right_logical(pos, i32(7))
        pcol = jnp.bitwise_and(pos, i32(CHUNK - 1))
        plsc.store_scatter(rid_v, [prow, pcol], rows, mask=in_stripe)
        plsc.store_scatter(cls_v, [pos], lbl - lo, mask=in_stripe)
        pc = plsc.all_reduce_population_count(in_stripe)
        return n + pc[0]

    def scan_body4(g4, n):
        for u in range(4):
            n = scan_body(g4 * i32(4) + i32(u), n)
        return n

    n = lax.fori_loop(jnp.int32(0), jnp.int32(GRP // 4), scan_body4,
                      jnp.int32(0))

    dummy = base + lo + lane
    for t in range(CHUNK // L):
        pos = n + i32(t * L) + lane
        prow = lax.shift_right_logical(pos, i32(7))
        pcol = jnp.bitwise_and(pos, i32(CHUNK - 1))
        plsc.store_scatter(rid_v, [prow, pcol], dummy)

    nch = lax.shift_right_logical(n + i32(CHUNK - 1), i32(7))

    def start_gather(j, rows_v, sem):
        pltpu.make_async_copy(ypred_hbm.at[rid_v.at[j]], rows_v, sem).start()

    def wait_gather(j, rows_v, sem):
        pltpu.make_async_copy(ypred_hbm.at[rid_v.at[j]], rows_v, sem).wait()

    def add_row(k_loc, cl, rows_v, sqs):
        out = list(sqs)
        for g in range(FG):
            v = rows_v[k_loc, pl.ds(g * L, L)]
            plsc.addupdate(acc_v.at[cl, pl.ds(g * L, L)], v)
            out[g % NSQ] = out[g % NSQ] + v * v
        plsc.addupdate(cnt_v.at[cl, pl.ds(0, L)], one_col)
        return tuple(out)

    def accumulate(j, rows_v):
        k0 = j * i32(CHUNK)
        kn = jnp.maximum(jnp.minimum(n - k0, i32(CHUNK)), i32(0))
        zc = (zvec, zvec, zvec, zvec)

        def pair_rows(p, sqs):
            k = p * i32(2)
            clv = cls_v[pl.ds(k0 + k, L)]
            sqs = add_row(k, clv[0], rows_v, sqs)
            sqs = add_row(k + 1, clv[1], rows_v, sqs)
            return sqs

        sqs = lax.fori_loop(jnp.int32(0), lax.shift_right_logical(kn, i32(1)),
                            pair_rows, zc)
        for q in range(NSQ):
            plsc.addupdate(sq_v.at[i32(q), pl.ds(0, L)], sqs[q])

        @pl.when(jnp.bitwise_and(kn, i32(1)) == 1)
        def _():
            k = kn - 1
            clv = cls_v[pl.ds(k0 + k, L)]
            tail = add_row(k, clv[0], rows_v, (zvec, zvec, zvec, zvec))
            for q in range(NSQ):
                plsc.addupdate(sq_v.at[i32(q), pl.ds(0, L)], tail[q])

    @pl.when(nch > 0)
    def _():
        start_gather(i32(0), rows_a, sem_a)

    def pair_body(jp, carry):
        j0 = jp * i32(2)
        j1 = j0 + 1

        wait_gather(j0, rows_a, sem_a)

        @pl.when(j1 < nch)
        def _():
            start_gather(j1, rows_b, sem_b)

        accumulate(j0, rows_a)

        @pl.when(j1 < nch)
        def _():
            wait_gather(j1, rows_b, sem_b)

            @pl.when(j1 + 1 < nch)
            def _():
                start_gather(j1 + 1, rows_a, sem_a)

            accumulate(j1, rows_b)

        return carry

    npair = lax.shift_right_logical(nch + 1, i32(1))
    lax.fori_loop(jnp.int32(0), npair, pair_body, jnp.int32(0))

    sqt = (sq_v[i32(0), pl.ds(0, L)] + sq_v[i32(1), pl.ds(0, L)]
           + sq_v[i32(2), pl.ds(0, L)] + sq_v[i32(3), pl.ds(0, L)])
    sq_v[i32(0), pl.ds(0, L)] = sqt

    pltpu.sync_copy(acc_v, sacc_hbm.at[c, pl.ds(lo, CPT)])
    pltpu.sync_copy(cnt_v, cacc_hbm.at[c, pl.ds(lo, CPT)])
    pltpu.sync_copy(sq_v.at[i32(0)], sqacc_hbm.at[c, s])


_Z = _np.int32(0)


def _combine_tc(centers_ref, sacc_ref, cacc_ref, sqacc_ref,
                upd_ref, loss_ref):
    S = sacc_ref[0] + sacc_ref[1]
    cnt = (cacc_ref[0] + cacc_ref[1])[:, 0:1]
    inv = ALPHA / (1.0 + cnt)
    cen = centers_ref[...]
    upd_ref[...] = cen * (1.0 - inv * cnt) + inv * S
    loss = (0.5 * jnp.sum(sqacc_ref[...])
            + 0.5 * jnp.sum(cnt * (cen * cen)) - jnp.sum(S * cen))
    loss_ref[...] = jnp.reshape(loss, (1, 1))


_combine_call = pl.pallas_call(
    _combine_tc,
    in_specs=[
        pl.BlockSpec((NUM_CLASSES, FEAT), lambda: (_Z, _Z)),
        pl.BlockSpec((NC, NUM_CLASSES, FEAT), lambda: (_Z, _Z, _Z)),
        pl.BlockSpec((NC, NUM_CLASSES, CPAD), lambda: (_Z, _Z, _Z)),
        pl.BlockSpec((NC, NS, L), lambda: (_Z, _Z, _Z)),
    ],
    out_specs=[
        pl.BlockSpec((NUM_CLASSES, FEAT), lambda: (_Z, _Z)),
        pl.BlockSpec((1, 1), lambda: (_Z, _Z)),
    ],
    out_shape=[
        jax.ShapeDtypeStruct((NUM_CLASSES, FEAT), jnp.float32),
        jax.ShapeDtypeStruct((1, 1), jnp.float32),
    ],
)


def kernel(y_true, y_pred, centers):
    labels = jnp.reshape(y_true.astype(jnp.int32), (NC, HALF))
    y_pred = y_pred.astype(jnp.float32)
    zrows = jnp.zeros((CPT, FEAT), jnp.float32)
    zcnt = jnp.zeros((CPT, CPAD), jnp.float32)
    sacc, cacc, sqacc = _segsum_sc(labels, y_pred, zrows, zcnt)
    upd, loss = _combine_call(centers, sacc, cacc, sqacc)
    return (loss[0, 0], centers, upd)


# speedup vs baseline: 1.3047x; 1.3047x over previous
import functools

import numpy as _np

import jax
import jax.numpy as jnp
from jax import lax
from jax.experimental import pallas as pl
from jax.experimental.pallas import tpu as pltpu
from jax.experimental.pallas import tpu_sc as plsc

NUM_CLASSES = 1024
FEAT = 256
ALPHA = 0.5
B = 16384

NC, NS = 2, 16
L = 16
HALF = B // NC
CPT = NUM_CLASSES // NS
CHUNK = 128
GRP = HALF // L
NCHK = HALF // CHUNK
LPAD = HALF + CHUNK + L
CPAD = 16
FG = FEAT // L
NSQ = 4

_mesh = plsc.VectorSubcoreMesh(core_axis_name="c", subcore_axis_name="s")


@functools.partial(
    pl.kernel,
    out_type=(
        jax.ShapeDtypeStruct((NC, NUM_CLASSES, FEAT), jnp.float32),
        jax.ShapeDtypeStruct((NC, NUM_CLASSES, CPAD), jnp.float32),
        jax.ShapeDtypeStruct((NC, NS, L), jnp.float32),
    ),
    mesh=_mesh,
    compiler_params=pltpu.CompilerParams(needs_layout_passes=False),
    scratch_types=[
        pltpu.VMEM((HALF,), jnp.int32),
        pltpu.VMEM((NCHK + 1, CHUNK), jnp.int32),
        pltpu.VMEM((LPAD,), jnp.int32),
        pltpu.VMEM((CHUNK, FEAT), jnp.float32),
        pltpu.VMEM((CHUNK, FEAT), jnp.float32),
        pltpu.VMEM((CPT, FEAT), jnp.float32),
        pltpu.VMEM((CPT, CPAD), jnp.float32),
        pltpu.VMEM((NSQ, L), jnp.float32),
        pltpu.SemaphoreType.DMA,
        pltpu.SemaphoreType.DMA,
    ],
)
def _segsum_sc(labels_hbm, ypred_hbm, zrows_hbm, zcnt_hbm,
               sacc_hbm, cacc_hbm, sqacc_hbm,
               lab_v, rid_v, cls_v, rows_a, rows_b, acc_v, cnt_v, sq_v,
               sem_a, sem_b):
    c = lax.axis_index("c")
    s = lax.axis_index("s")
    i32 = jnp.int32
    lo = s * i32(CPT)
    base = c * i32(HALF)

    pltpu.sync_copy(zrows_hbm, acc_v)
    pltpu.sync_copy(zcnt_hbm, cnt_v)
    pltpu.sync_copy(labels_hbm.at[c], lab_v)

    lane = lax.iota(jnp.int32, L)
    one_col = jnp.where(lane == 0, jnp.float32(1.0), jnp.float32(0.0))
    zvec = jnp.zeros((L,), jnp.float32)
    for q in range(NSQ):
        sq_v[i32(q), pl.ds(0, L)] = zvec

    def scan_body(g, n):
        lbl = lab_v[pl.ds(g * i32(L), L)]
        in_stripe = jnp.logical_and(lbl >= lo, lbl < lo + CPT)
        rows = base + g * i32(L) + lane
        inc = plsc.cumsum(in_stripe.astype(jnp.int32))
        pos = jnp.maximum(n + inc - 1, i32(0))
        prow = lax.shift_right_logical(pos, i32(7))
        pcol = jnp.bitwise_and(pos, i32(CHUNK - 1))
        plsc.store_scatter(rid_v, [prow, pcol], rows, mask=in_stripe)
        plsc.store_scatter(cls_v, [pos], lbl - lo, mask=in_stripe)
        pc = plsc.all_reduce_population_count(in_stripe)
        return n + pc[0]

    def scan_body4(g4, n):
        for u in range(4):
            n = scan_body(g4 * i32(4) + i32(u), n)
        return n

    n = lax.fori_loop(jnp.int32(0), jnp.int32(GRP // 4), scan_body4,
                      jnp.int32(0))

    dummy = base + lo + lane
    for t in range(CHUNK // L):
        pos = n + i32(t * L) + lane
        prow = lax.shift_right_logical(pos, i32(7))
        pcol = jnp.bitwise_and(pos, i32(CHUNK - 1))
        plsc.store_scatter(rid_v, [prow, pcol], dummy)

    nch = lax.shift_right_logical(n + i32(CHUNK - 1), i32(7))

    def start_gather(j, rows_v, sem):
        pltpu.make_async_copy(ypred_hbm.at[rid_v.at[j]], rows_v, sem).start()

    def wait_gather(j, rows_v, sem):
        pltpu.make_async_copy(ypred_hbm.at[rid_v.at[j]], rows_v, sem).wait()

    def add_row(k_loc, cl, rows_v, sqs):
        out = list(sqs)
        for g in range(FG):
            v = rows_v[k_loc, pl.ds(g * L, L)]
            plsc.addupdate(acc_v.at[cl, pl.ds(g * L, L)], v)
            out[g % NSQ] = out[g % NSQ] + v * v
        plsc.addupdate(cnt_v.at[cl, pl.ds(0, L)], one_col)
        return tuple(out)

    def accumulate(j, rows_v):
        k0 = j * i32(CHUNK)
        kn = jnp.maximum(jnp.minimum(n - k0, i32(CHUNK)), i32(0))
        zc = (zvec, zvec, zvec, zvec)

        def pair_rows(p, sqs):
            k = p * i32(2)
            clv = cls_v[pl.ds(k0 + k, L)]
            sqs = add_row(k, clv[0], rows_v, sqs)
            sqs = add_row(k + 1, clv[1], rows_v, sqs)
            return sqs

        sqs = plsc.parallel_loop(jnp.int32(0),
                                 lax.shift_right_logical(kn, i32(1)),
                                 jnp.int32(1), carry=zc)(pair_rows)
        for q in range(NSQ):
            plsc.addupdate(sq_v.at[i32(q), pl.ds(0, L)], sqs[q])

        @pl.when(jnp.bitwise_and(kn, i32(1)) == 1)
        def _():
            k = kn - 1
            clv = cls_v[pl.ds(k0 + k, L)]
            tail = add_row(k, clv[0], rows_v, (zvec, zvec, zvec, zvec))
            for q in range(NSQ):
                plsc.addupdate(sq_v.at[i32(q), pl.ds(0, L)], tail[q])

    @pl.when(nch > 0)
    def _():
        start_gather(i32(0), rows_a, sem_a)

    def pair_body(jp, carry):
        j0 = jp * i32(2)
        j1 = j0 + 1

        wait_gather(j0, rows_a, sem_a)

        @pl.when(j1 < nch)
        def _():
            start_gather(j1, rows_b, sem_b)

        accumulate(j0, rows_a)

        @pl.when(j1 < nch)
        def _():
            wait_gather(j1, rows_b, sem_b)

            @pl.when(j1 + 1 < nch)
            def _():
                start_gather(j1 + 1, rows_a, sem_a)

            accumulate(j1, rows_b)

        return carry

    npair = lax.shift_right_logical(nch + 1, i32(1))
    lax.fori_loop(jnp.int32(0), npair, pair_body, jnp.int32(0))

    sqt = (sq_v[i32(0), pl.ds(0, L)] + sq_v[i32(1), pl.ds(0, L)]
           + sq_v[i32(2), pl.ds(0, L)] + sq_v[i32(3), pl.ds(0, L)])
    sq_v[i32(0), pl.ds(0, L)] = sqt

    pltpu.sync_copy(acc_v, sacc_hbm.at[c, pl.ds(lo, CPT)])
    pltpu.sync_copy(cnt_v, cacc_hbm.at[c, pl.ds(lo, CPT)])
    pltpu.sync_copy(sq_v.at[i32(0)], sqacc_hbm.at[c, s])


_Z = _np.int32(0)


def _combine_tc(centers_ref, sacc_ref, cacc_ref, sqacc_ref,
                upd_ref, loss_ref):
    S = sacc_ref[0] + sacc_ref[1]
    cnt = (cacc_ref[0] + cacc_ref[1])[:, 0:1]
    inv = ALPHA / (1.0 + cnt)
    cen = centers_ref[...]
    upd_ref[...] = cen * (1.0 - inv * cnt) + inv * S
    loss = (0.5 * jnp.sum(sqacc_ref[...])
            + 0.5 * jnp.sum(cnt * (cen * cen)) - jnp.sum(S * cen))
    loss_ref[...] = jnp.reshape(loss, (1, 1))


_combine_call = pl.pallas_call(
    _combine_tc,
    in_specs=[
        pl.BlockSpec((NUM_CLASSES, FEAT), lambda: (_Z, _Z)),
        pl.BlockSpec((NC, NUM_CLASSES, FEAT), lambda: (_Z, _Z, _Z)),
        pl.BlockSpec((NC, NUM_CLASSES, CPAD), lambda: (_Z, _Z, _Z)),
        pl.BlockSpec((NC, NS, L), lambda: (_Z, _Z, _Z)),
    ],
    out_specs=[
        pl.BlockSpec((NUM_CLASSES, FEAT), lambda: (_Z, _Z)),
        pl.BlockSpec((1, 1), lambda: (_Z, _Z)),
    ],
    out_shape=[
        jax.ShapeDtypeStruct((NUM_CLASSES, FEAT), jnp.float32),
        jax.ShapeDtypeStruct((1, 1), jnp.float32),
    ],
)


def kernel(y_true, y_pred, centers):
    labels = jnp.reshape(y_true.astype(jnp.int32), (NC, HALF))
    y_pred = y_pred.astype(jnp.float32)
    zrows = jnp.zeros((CPT, FEAT), jnp.float32)
    zcnt = jnp.zeros((CPT, CPAD), jnp.float32)
    sacc, cacc, sqacc = _segsum_sc(labels, y_pred, zrows, zcnt)
    upd, loss = _combine_call(centers, sacc, cacc, sqacc)
    return (loss[0, 0], centers, upd)


# speedup vs baseline: 1.5022x; 1.1514x over previous
import functools

import numpy as _np

import jax
import jax.numpy as jnp
from jax import lax
from jax.experimental import pallas as pl
from jax.experimental.pallas import tpu as pltpu
from jax.experimental.pallas import tpu_sc as plsc

NUM_CLASSES = 1024
FEAT = 256
ALPHA = 0.5
B = 16384

NC, NS = 2, 16
L = 16
HALF = B // NC
CPT = NUM_CLASSES // NS
CHUNK = 128
GRP = HALF // L
NCHK = HALF // CHUNK
LPAD = HALF + CHUNK + L
CPAD = 16
FG = FEAT // L
NSQ = 4

_mesh = plsc.VectorSubcoreMesh(core_axis_name="c", subcore_axis_name="s")


@functools.partial(
    pl.kernel,
    out_type=(
        jax.ShapeDtypeStruct((NC, NUM_CLASSES, FEAT), jnp.float32),
        jax.ShapeDtypeStruct((NC, NUM_CLASSES, CPAD), jnp.float32),
        jax.ShapeDtypeStruct((NC, NS, L), jnp.float32),
    ),
    mesh=_mesh,
    compiler_params=pltpu.CompilerParams(needs_layout_passes=False),
    scratch_types=[
        pltpu.VMEM((HALF,), jnp.int32),
        pltpu.VMEM((NCHK + 1, CHUNK), jnp.int32),
        pltpu.VMEM((LPAD,), jnp.int32),
        pltpu.VMEM((CHUNK, FEAT), jnp.float32),
        pltpu.VMEM((CHUNK, FEAT), jnp.float32),
        pltpu.VMEM((CPT, FEAT), jnp.float32),
        pltpu.VMEM((CPT, CPAD), jnp.float32),
        pltpu.VMEM((NSQ, L), jnp.float32),
        pltpu.SemaphoreType.DMA,
        pltpu.SemaphoreType.DMA,
    ],
)
def _segsum_sc(labels_hbm, ypred_hbm,
               sacc_hbm, cacc_hbm, sqacc_hbm,
               lab_v, rid_v, cls_v, rows_a, rows_b, acc_v, cnt_v, sq_v,
               sem_a, sem_b):
    c = lax.axis_index("c")
    s = lax.axis_index("s")
    i32 = jnp.int32
    lo = s * i32(CPT)
    base = c * i32(HALF)

    pltpu.sync_copy(labels_hbm.at[c], lab_v)

    lane = lax.iota(jnp.int32, L)
    one_col = jnp.where(lane == 0, jnp.float32(1.0), jnp.float32(0.0))
    zvec = jnp.zeros((L,), jnp.float32)
    for q in range(NSQ):
        sq_v[i32(q), pl.ds(0, L)] = zvec

    @plsc.parallel_loop(jnp.int32(0), jnp.int32(CPT), jnp.int32(1))
    def _zero_row(r):
        for g in range(FG):
            acc_v[r, pl.ds(g * L, L)] = zvec
        cnt_v[r, pl.ds(0, L)] = zvec

    def scan_body(g, n):
        lbl = lab_v[pl.ds(g * i32(L), L)]
        in_stripe = jnp.logical_and(lbl >= lo, lbl < lo + CPT)
        rows = base + g * i32(L) + lane
        inc = plsc.cumsum(in_stripe.astype(jnp.int32))
        pos = jnp.maximum(n + inc - 1, i32(0))
        prow = lax.shift_right_logical(pos, i32(7))
        pcol = jnp.bitwise_and(pos, i32(CHUNK - 1))
        plsc.store_scatter(rid_v, [prow, pcol], rows, mask=in_stripe)
        plsc.store_scatter(cls_v, [pos], lbl - lo, mask=in_stripe)
        pc = plsc.all_reduce_population_count(in_stripe)
        return n + pc[0]

    n = plsc.parallel_loop(jnp.int32(0), jnp.int32(GRP), jnp.int32(1),
                           unroll=4, carry=jnp.int32(0))(scan_body)

    dummy = base + lo + lane
    for t in range(CHUNK // L):
        pos = n + i32(t * L) + lane
        prow = lax.shift_right_logical(pos, i32(7))
        pcol = jnp.bitwise_and(pos, i32(CHUNK - 1))
        plsc.store_scatter(rid_v, [prow, pcol], dummy)

    nch = lax.shift_right_logical(n + i32(CHUNK - 1), i32(7))

    def start_gather(j, rows_v, sem):
        pltpu.make_async_copy(ypred_hbm.at[rid_v.at[j]], rows_v, sem).start()

    def wait_gather(j, rows_v, sem):
        pltpu.make_async_copy(ypred_hbm.at[rid_v.at[j]], rows_v, sem).wait()

    def add_row(k_loc, cl, rows_v, sqs):
        out = list(sqs)
        for g in range(FG):
            v = rows_v[k_loc, pl.ds(g * L, L)]
            plsc.addupdate(acc_v.at[cl, pl.ds(g * L, L)], v)
            out[g % NSQ] = out[g % NSQ] + v * v
        plsc.addupdate(cnt_v.at[cl, pl.ds(0, L)], one_col)
        return tuple(out)

    def accumulate(j, rows_v):
        k0 = j * i32(CHUNK)
        kn = jnp.maximum(jnp.minimum(n - k0, i32(CHUNK)), i32(0))
        zc = (zvec, zvec, zvec, zvec)

        def pair_rows(p, sqs):
            k = p * i32(2)
            clv = cls_v[pl.ds(k0 + k, L)]
            sqs = add_row(k, clv[0], rows_v, sqs)
            sqs = add_row(k + 1, clv[1], rows_v, sqs)
            return sqs

        sqs = plsc.parallel_loop(jnp.int32(0),
                                 lax.shift_right_logical(kn, i32(1)),
                                 jnp.int32(1), unroll=2, carry=zc)(pair_rows)
        for q in range(NSQ):
            plsc.addupdate(sq_v.at[i32(q), pl.ds(0, L)], sqs[q])

        @pl.when(jnp.bitwise_and(kn, i32(1)) == 1)
        def _():
            k = kn - 1
            clv = cls_v[pl.ds(k0 + k, L)]
            tail = add_row(k, clv[0], rows_v, (zvec, zvec, zvec, zvec))
            for q in range(NSQ):
                plsc.addupdate(sq_v.at[i32(q), pl.ds(0, L)], tail[q])

    @pl.when(nch > 0)
    def _():
        start_gather(i32(0), rows_a, sem_a)

    def pair_body(jp, carry):
        j0 = jp * i32(2)
        j1 = j0 + 1

        wait_gather(j0, rows_a, sem_a)

        @pl.when(j1 < nch)
        def _():
            start_gather(j1, rows_b, sem_b)

        accumulate(j0, rows_a)

        @pl.when(j1 < nch)
        def _():
            wait_gather(j1, rows_b, sem_b)

            @pl.when(j1 + 1 < nch)
            def _():
                start_gather(j1 + 1, rows_a, sem_a)

            accumulate(j1, rows_b)

        return carry

    npair = lax.shift_right_logical(nch + 1, i32(1))
    lax.fori_loop(jnp.int32(0), npair, pair_body, jnp.int32(0))

    sqt = (sq_v[i32(0), pl.ds(0, L)] + sq_v[i32(1), pl.ds(0, L)]
           + sq_v[i32(2), pl.ds(0, L)] + sq_v[i32(3), pl.ds(0, L)])
    sq_v[i32(0), pl.ds(0, L)] = sqt

    pltpu.sync_copy(acc_v, sacc_hbm.at[c, pl.ds(lo, CPT)])
    pltpu.sync_copy(cnt_v, cacc_hbm.at[c, pl.ds(lo, CPT)])
    pltpu.sync_copy(sq_v.at[i32(0)], sqacc_hbm.at[c, s])


_Z = _np.int32(0)


def _combine_tc(centers_ref, sacc_ref, cacc_ref, sqacc_ref,
                upd_ref, loss_ref):
    S = sacc_ref[0] + sacc_ref[1]
    cnt = (cacc_ref[0] + cacc_ref[1])[:, 0:1]
    inv = ALPHA / (1.0 + cnt)
    cen = centers_ref[...]
    upd_ref[...] = cen * (1.0 - inv * cnt) + inv * S
    loss = (0.5 * jnp.sum(sqacc_ref[...])
            + 0.5 * jnp.sum(cnt * (cen * cen)) - jnp.sum(S * cen))
    loss_ref[...] = jnp.reshape(loss, (1, 1))


_combine_call = pl.pallas_call(
    _combine_tc,
    in_specs=[
        pl.BlockSpec((NUM_CLASSES, FEAT), lambda: (_Z, _Z)),
        pl.BlockSpec((NC, NUM_CLASSES, FEAT), lambda: (_Z, _Z, _Z)),
        pl.BlockSpec((NC, NUM_CLASSES, CPAD), lambda: (_Z, _Z, _Z)),
        pl.BlockSpec((NC, NS, L), lambda: (_Z, _Z, _Z)),
    ],
    out_specs=[
        pl.BlockSpec((NUM_CLASSES, FEAT), lambda: (_Z, _Z)),
        pl.BlockSpec((1, 1), lambda: (_Z, _Z)),
    ],
    out_shape=[
        jax.ShapeDtypeStruct((NUM_CLASSES, FEAT), jnp.float32),
        jax.ShapeDtypeStruct((1, 1), jnp.float32),
    ],
)


def kernel(y_true, y_pred, centers):
    labels = jnp.reshape(y_true.astype(jnp.int32), (NC, HALF))
    y_pred = y_pred.astype(jnp.float32)
    sacc, cacc, sqacc = _segsum_sc(labels, y_pred)
    upd, loss = _combine_call(centers, sacc, cacc, sqacc)
    return (loss[0, 0], centers, upd)


# speedup vs baseline: 1.5022x; 1.0001x over previous
import functools

import numpy as _np

import jax
import jax.numpy as jnp
from jax import lax
from jax.experimental import pallas as pl
from jax.experimental.pallas import tpu as pltpu
from jax.experimental.pallas import tpu_sc as plsc

NUM_CLASSES = 1024
FEAT = 256
ALPHA = 0.5
B = 16384

NC, NS = 2, 16
L = 16
HALF = B // NC
CPT = NUM_CLASSES // NS
CHUNK = 128
GRP = HALF // L
NCHK = HALF // CHUNK
LPAD = HALF + CHUNK + L
CPAD = 16
FG = FEAT // L
NSQ = 4

_mesh = plsc.VectorSubcoreMesh(core_axis_name="c", subcore_axis_name="s")


@functools.partial(
    pl.kernel,
    out_type=(
        jax.ShapeDtypeStruct((NC, NUM_CLASSES, FEAT), jnp.float32),
        jax.ShapeDtypeStruct((NC, NUM_CLASSES, CPAD), jnp.float32),
        jax.ShapeDtypeStruct((NC, NS, L), jnp.float32),
    ),
    mesh=_mesh,
    compiler_params=pltpu.CompilerParams(needs_layout_passes=False),
    scratch_types=[
        pltpu.VMEM((HALF,), jnp.int32),
        pltpu.VMEM((NCHK + 1, CHUNK), jnp.int32),
        pltpu.VMEM((LPAD,), jnp.int32),
        pltpu.VMEM((CHUNK, FEAT), jnp.float32),
        pltpu.VMEM((CHUNK, FEAT), jnp.float32),
        pltpu.VMEM((CPT, FEAT), jnp.float32),
        pltpu.VMEM((CPT, CPAD), jnp.float32),
        pltpu.VMEM((NSQ, L), jnp.float32),
        pltpu.SemaphoreType.DMA,
        pltpu.SemaphoreType.DMA,
    ],
)
def _segsum_sc(labels_hbm, ypred_hbm,
               sacc_hbm, cacc_hbm, sqacc_hbm,
               lab_v, rid_v, cls_v, rows_a, rows_b, acc_v, cnt_v, sq_v,
               sem_a, sem_b):
    c = lax.axis_index("c")
    s = lax.axis_index("s")
    i32 = jnp.int32
    lo = s * i32(CPT)
    base = c * i32(HALF)

    pltpu.sync_copy(labels_hbm.at[c], lab_v)

    lane = lax.iota(jnp.int32, L)
    one_col = jnp.where(lane == 0, jnp.float32(1.0), jnp.float32(0.0))
    zvec = jnp.zeros((L,), jnp.float32)
    for q in range(NSQ):
        sq_v[i32(q), pl.ds(0, L)] = zvec

    @plsc.parallel_loop(jnp.int32(0), jnp.int32(CPT), jnp.int32(1))
    def _zero_row(r):
        for g in range(FG):
            acc_v[r, pl.ds(g * L, L)] = zvec
        cnt_v[r, pl.ds(0, L)] = zvec

    def scan_body(g, n):
        lbl = lab_v[pl.ds(g * i32(L), L)]
        in_stripe = jnp.logical_and(lbl >= lo, lbl < lo + CPT)
        rows = base + g * i32(L) + lane
        inc = plsc.cumsum(in_stripe.astype(jnp.int32))
        pos = jnp.maximum(n + inc - 1, i32(0))
        prow = lax.shift_right_logical(pos, i32(7))
        pcol = jnp.bitwise_and(pos, i32(CHUNK - 1))
        plsc.store_scatter(rid_v, [prow, pcol], rows, mask=in_stripe)
        plsc.store_scatter(cls_v, [pos], lbl - lo, mask=in_stripe)
        pc = plsc.all_reduce_population_count(in_stripe)
        return n + pc[0]

    n = plsc.parallel_loop(jnp.int32(0), jnp.int32(GRP), jnp.int32(1),
                           unroll=4, carry=jnp.int32(0))(scan_body)

    dummy = base + lo + lane
    for t in range(CHUNK // L):
        pos = n + i32(t * L) + lane
        prow = lax.shift_right_logical(pos, i32(7))
        pcol = jnp.bitwise_and(pos, i32(CHUNK - 1))
        plsc.store_scatter(rid_v, [prow, pcol], dummy)

    nch = lax.shift_right_logical(n + i32(CHUNK - 1), i32(7))

    def start_gather(j, rows_v, sem):
        pltpu.make_async_copy(ypred_hbm.at[rid_v.at[j]], rows_v, sem).start()

    def wait_gather(j, rows_v, sem):
        pltpu.make_async_copy(ypred_hbm.at[rid_v.at[j]], rows_v, sem).wait()

    def add_row(k_loc, cl, rows_v, sqs):
        out = list(sqs)
        for g in range(FG):
            v = rows_v[k_loc, pl.ds(g * L, L)]
            plsc.addupdate(acc_v.at[cl, pl.ds(g * L, L)], v)
            out[g % NSQ] = out[g % NSQ] + v * v
        plsc.addupdate(cnt_v.at[cl, pl.ds(0, L)], one_col)
        return tuple(out)

    def accumulate(j, rows_v):
        k0 = j * i32(CHUNK)
        kn = jnp.maximum(jnp.minimum(n - k0, i32(CHUNK)), i32(0))
        zc = (zvec, zvec, zvec, zvec)

        def pair_rows(p, sqs):
            k = p * i32(2)
            clv = cls_v[pl.ds(k0 + k, L)]
            sqs = add_row(k, clv[0], rows_v, sqs)
            sqs = add_row(k + 1, clv[1], rows_v, sqs)
            return sqs

        sqs = plsc.parallel_loop(jnp.int32(0),
                                 lax.shift_right_logical(kn, i32(1)),
                                 jnp.int32(1), unroll=2, carry=zc)(pair_rows)
        for q in range(NSQ):
            plsc.addupdate(sq_v.at[i32(q), pl.ds(0, L)], sqs[q])

        @pl.when(jnp.bitwise_and(kn, i32(1)) == 1)
        def _():
            k = kn - 1
            clv = cls_v[pl.ds(k0 + k, L)]
            tail = add_row(k, clv[0], rows_v, (zvec, zvec, zvec, zvec))
            for q in range(NSQ):
                plsc.addupdate(sq_v.at[i32(q), pl.ds(0, L)], tail[q])

    @pl.when(nch > 0)
    def _():
        start_gather(i32(0), rows_a, sem_a)

    def pair_body(jp, carry):
        j0 = jp * i32(2)
        j1 = j0 + 1

        wait_gather(j0, rows_a, sem_a)

        @pl.when(j1 < nch)
        def _():
            start_gather(j1, rows_b, sem_b)

        accumulate(j0, rows_a)

        @pl.when(j1 < nch)
        def _():
            wait_gather(j1, rows_b, sem_b)

            @pl.when(j1 + 1 < nch)
            def _():
                start_gather(j1 + 1, rows_a, sem_a)

            accumulate(j1, rows_b)

        return carry

    npair = lax.shift_right_logical(nch + 1, i32(1))
    lax.fori_loop(jnp.int32(0), npair, pair_body, jnp.int32(0))

    sqt = (sq_v[i32(0), pl.ds(0, L)] + sq_v[i32(1), pl.ds(0, L)]
           + sq_v[i32(2), pl.ds(0, L)] + sq_v[i32(3), pl.ds(0, L)])
    sq_v[i32(0), pl.ds(0, L)] = sqt

    out1 = pltpu.make_async_copy(acc_v, sacc_hbm.at[c, pl.ds(lo, CPT)], sem_a)
    out2 = pltpu.make_async_copy(cnt_v, cacc_hbm.at[c, pl.ds(lo, CPT)], sem_b)
    out1.start()
    out2.start()
    pltpu.sync_copy(sq_v.at[i32(0)], sqacc_hbm.at[c, s])
    out1.wait()
    out2.wait()


_Z = _np.int32(0)


def _combine_tc(centers_ref, sacc_ref, cacc_ref, sqacc_ref,
                upd_ref, loss_ref):
    S = sacc_ref[0] + sacc_ref[1]
    cnt = (cacc_ref[0] + cacc_ref[1])[:, 0:1]
    inv = ALPHA / (1.0 + cnt)
    cen = centers_ref[...]
    upd_ref[...] = cen * (1.0 - inv * cnt) + inv * S
    loss = (0.5 * jnp.sum(sqacc_ref[...])
            + 0.5 * jnp.sum(cnt * (cen * cen)) - jnp.sum(S * cen))
    loss_ref[...] = jnp.reshape(loss, (1, 1))


_combine_call = pl.pallas_call(
    _combine_tc,
    in_specs=[
        pl.BlockSpec((NUM_CLASSES, FEAT), lambda: (_Z, _Z)),
        pl.BlockSpec((NC, NUM_CLASSES, FEAT), lambda: (_Z, _Z, _Z)),
        pl.BlockSpec((NC, NUM_CLASSES, CPAD), lambda: (_Z, _Z, _Z)),
        pl.BlockSpec((NC, NS, L), lambda: (_Z, _Z, _Z)),
    ],
    out_specs=[
        pl.BlockSpec((NUM_CLASSES, FEAT), lambda: (_Z, _Z)),
        pl.BlockSpec((1, 1), lambda: (_Z, _Z)),
    ],
    out_shape=[
        jax.ShapeDtypeStruct((NUM_CLASSES, FEAT), jnp.float32),
        jax.ShapeDtypeStruct((1, 1), jnp.float32),
    ],
)


def kernel(y_true, y_pred, centers):
    labels = jnp.reshape(y_true.astype(jnp.int32), (NC, HALF))
    y_pred = y_pred.astype(jnp.float32)
    sacc, cacc, sqacc = _segsum_sc(labels, y_pred)
    upd, loss = _combine_call(centers, sacc, cacc, sqacc)
    return (loss[0, 0], centers, upd)
